# baseline (speedup 1.0000x reference)
import functools

import jax
import jax.numpy as jnp
from jax import lax
from jax.experimental import pallas as pl
from jax.experimental.pallas import tpu as pltpu
from jax.experimental.pallas import tpu_sc as plsc

_L = 16
_W = 128
_BW = 256
_BSH = 8
_RUNS_MAX = 544


def _stage_pair(uk, ik, u_tab, i_tab):
    B = uk.shape[0]
    D, V = u_tab.shape
    info = plsc.get_sparse_core_info()
    NC, NS = info.num_cores, info.num_subcores
    NW = NC * NS
    n = B // NW
    NH = D // _L
    mesh = plsc.VectorSubcoreMesh(core_axis_name="c", subcore_axis_name="s")

    @functools.partial(
        pl.kernel,
        out_type=(jax.ShapeDtypeStruct((B, _W), jnp.float32),
                  jax.ShapeDtypeStruct((B, _W), jnp.float32)),
        mesh=mesh,
        scratch_types=[
            pltpu.VMEM((n,), jnp.int32),
            pltpu.VMEM((n,), jnp.int32),
            pltpu.VMEM((_RUNS_MAX,), jnp.int32),
            pltpu.VMEM((D, _BW), jnp.float32),
            pltpu.VMEM((D, _BW), jnp.float32),
            pltpu.VMEM((D, _BW), jnp.float32),
            pltpu.VMEM((n, _W), jnp.float32),
            pltpu.SemaphoreType.DMA,
        ],
        compiler_params=pltpu.CompilerParams(
            needs_layout_passes=False, use_tc_tiling_on_sc=True,
            disable_bounds_checks=True),
    )
    def stage_kernel(uk_hbm, ik_hbm, u_tab_hbm, i_tab_hbm, u_gath, i_gath,
                     keys_v, tgs_v, runs_v, bb0, bb1, bb2,
                     staged, semA):
        bufs = (bb0, bb1, bb2)
        vpad = -(-V // _W) * _W
        wid = lax.axis_index("s") * NC + lax.axis_index("c")
        base = wid * n
        lanes = lax.iota(jnp.int32, _L)

        def splat(x):
            return jnp.full((_L,), x, jnp.int32)

        def sread(ref1d, e):
            v = plsc.load_gather(ref1d, [splat(e)])
            return lax.reduce_max(v, (0,))

        def side(keys_hbm, tab, gath):
            pltpu.sync_copy(keys_hbm.at[pl.ds(base, n)], keys_v)

            for v in range(n // _L):
                k = keys_v[pl.ds(v * _L, _L)]
                tgs_v[pl.ds(v * _L, _L)] = lax.shift_right_logical(k, _BSH)
            for v in range(_RUNS_MAX // _L):
                runs_v[pl.ds(v * _L, _L)] = splat(n)

            def pa(v, cnt_vec):
                tg = tgs_v[pl.ds(v * _L, _L)]
                pidx = jnp.maximum(v * _L - 1 + lanes, 0)
                prev = plsc.load_gather(tgs_v, [pidx])
                m = tg != prev
                m = jnp.logical_or(m, jnp.logical_and(v == 0, lanes == 0))
                mi = m.astype(jnp.int32)
                excl = plsc.cumsum(mi) - mi
                plsc.store_scatter(runs_v, [cnt_vec + excl], v * _L + lanes,
                                   mask=m)
                return cnt_vec + plsc.all_reduce_population_count(m)
            cnt_vec = lax.fori_loop(0, n // _L, pa, jnp.zeros((_L,), jnp.int32))
            cnt = lax.reduce_max(cnt_vec, (0,))
            cnt2 = (cnt + 1) // 2

            def fetch(j, blk, sem):
                e0 = jnp.minimum(sread(runs_v, j), n - 1)
                tg = sread(tgs_v, e0)
                off = pl.multiple_of(
                    jnp.minimum(tg * _BW, vpad - _BW), _W)
                return pltpu.async_copy(tab.at[:, pl.ds(off, _BW)], blk, sem)

            def extract(j, blk):
                e0 = sread(runs_v, j)
                e1 = jnp.minimum(sread(runs_v, j + 1), n)

                def elem(e, c):
                    kv = plsc.load_gather(keys_v, [splat(e)])
                    boff = jnp.minimum(
                        lax.shift_right_logical(kv, _BSH) * _BW, vpad - _BW)
                    lane = kv - boff
                    for h in range(NH):
                        v = plsc.load_gather(blk, [lanes + _L * h, lane])
                        staged[e, pl.ds(_L * h, _L)] = v
                    return c
                lax.fori_loop(e0, e1, elem, 0)

            NG = 3

            def kb(k, c):
                j = NG * k
                hs = [fetch(j + t, bufs[t], semA) for t in range(NG)]
                for t in range(NG):
                    hs[t].wait()
                for t in range(NG):
                    extract(j + t, bufs[t])
                return c
            lax.fori_loop(0, (cnt + NG - 1) // NG, kb, 0)
            pltpu.sync_copy(staged, gath.at[pl.ds(base, n)])

        side(uk_hbm, u_tab_hbm, u_gath)
        side(ik_hbm, i_tab_hbm, i_gath)

    return stage_kernel(uk, ik, u_tab, i_tab)


def _dot(u_gath, i_gath, inv_u, inv_i, D):
    B = u_gath.shape[0]
    info = plsc.get_sparse_core_info()
    NC, NS = info.num_cores, info.num_subcores
    NW = NC * NS
    n = B // NW
    half = n // 2
    mesh = plsc.VectorSubcoreMesh(core_axis_name="c", subcore_axis_name="s")

    @functools.partial(
        pl.kernel,
        out_type=jax.ShapeDtypeStruct((B,), jnp.float32),
        mesh=mesh,
        scratch_types=[
            pltpu.VMEM((half,), jnp.int32),
            pltpu.VMEM((half,), jnp.int32),
            pltpu.VMEM((half, _W), jnp.float32),
            pltpu.VMEM((half, _W), jnp.float32),
            pltpu.VMEM((n,), jnp.float32),
            pltpu.SemaphoreType.DMA,
            pltpu.SemaphoreType.DMA,
        ],
        compiler_params=pltpu.CompilerParams(
            needs_layout_passes=False, use_tc_tiling_on_sc=True,
            disable_bounds_checks=True),
    )
    def dot_kernel(u_gath_hbm, i_gath_hbm, inv_u_hbm, inv_i_hbm, out_hbm,
                   iu_v, ii_v, u_buf, i_buf, out_v, semu, semi):
        wid = lax.axis_index("s") * NC + lax.axis_index("c")
        base = wid * n
        lanes = lax.iota(jnp.int32, _L)

        for hb in range(2):
            off = base + hb * half
            pltpu.sync_copy(inv_u_hbm.at[pl.ds(off, half)], iu_v)
            pltpu.sync_copy(inv_i_hbm.at[pl.ds(off, half)], ii_v)
            cu = pltpu.async_copy(u_gath_hbm.at[iu_v], u_buf, semu)
            ci = pltpu.async_copy(i_gath_hbm.at[ii_v], i_buf, semi)
            cu.wait()
            ci.wait()

            def group(g, c):
                rows = g * _L + lanes
                acc = jnp.zeros((_L,), jnp.float32)
                for d in range(D):
                    col = jnp.full((_L,), d, jnp.int32)
                    uu = plsc.load_gather(u_buf, [rows, col])
                    ii = plsc.load_gather(i_buf, [rows, col])
                    acc = acc + uu * ii
                out_v[pl.ds(hb * half + g * _L, _L)] = acc
                return c
            lax.fori_loop(0, half // _L, group, 0)
        pltpu.sync_copy(out_v, out_hbm.at[pl.ds(base, n)])

    return dot_kernel(u_gath, i_gath, inv_u, inv_i)


def kernel(u_id, i_id, user_emb, item_emb, user_bias, item_bias):
    del user_bias, item_bias
    u_id = u_id.astype(jnp.int32)
    i_id = i_id.astype(jnp.int32)
    B = u_id.shape[0]
    D = user_emb.shape[1]
    pos = lax.iota(jnp.int32, B)
    uk, up = lax.sort((u_id, pos), num_keys=1)
    ik, ip = lax.sort((i_id, pos), num_keys=1)
    inv_u = lax.sort((up, pos), num_keys=1)[1]
    inv_i = lax.sort((ip, pos), num_keys=1)[1]
    u_gath, i_gath = _stage_pair(uk, ik, user_emb.T, item_emb.T)
    return _dot(u_gath, i_gath, inv_u, inv_i, D)


# speedup vs baseline: 1.0697x; 1.0697x over previous
import functools

import jax
import jax.numpy as jnp
from jax import lax
from jax.experimental import pallas as pl
from jax.experimental.pallas import tpu as pltpu
from jax.experimental.pallas import tpu_sc as plsc

_L = 16
_W = 128
_BW = 128
_BSH = 7
_RUNS_MAX = 544


def _stage_pair(uk, ik, u_tab, i_tab):
    B = uk.shape[0]
    D, V = u_tab.shape
    info = plsc.get_sparse_core_info()
    NC, NS = info.num_cores, info.num_subcores
    NW = NC * NS
    n = B // NW
    NH = D // _L
    mesh = plsc.VectorSubcoreMesh(core_axis_name="c", subcore_axis_name="s")

    @functools.partial(
        pl.kernel,
        out_type=(jax.ShapeDtypeStruct((B, _W), jnp.float32),
                  jax.ShapeDtypeStruct((B, _W), jnp.float32)),
        mesh=mesh,
        scratch_types=[
            pltpu.VMEM((n,), jnp.int32),
            pltpu.VMEM((n,), jnp.int32),
            pltpu.VMEM((_RUNS_MAX,), jnp.int32),
            pltpu.VMEM((D, _BW), jnp.float32),
            pltpu.VMEM((D, _BW), jnp.float32),
            pltpu.VMEM((D, _BW), jnp.float32),
            pltpu.VMEM((D, _BW), jnp.float32),
            pltpu.VMEM((D, _BW), jnp.float32),
            pltpu.VMEM((D, _BW), jnp.float32),
            pltpu.VMEM((n, _W), jnp.float32),
            pltpu.SemaphoreType.DMA,
        ],
        compiler_params=pltpu.CompilerParams(
            needs_layout_passes=False, use_tc_tiling_on_sc=True,
            disable_bounds_checks=True),
    )
    def stage_kernel(uk_hbm, ik_hbm, u_tab_hbm, i_tab_hbm, u_gath, i_gath,
                     keys_v, tgs_v, runs_v, bb0, bb1, bb2, bb3, bb4, bb5,
                     staged, semA):
        bufs = (bb0, bb1, bb2, bb3, bb4, bb5)
        vpad = -(-V // _W) * _W
        wid = lax.axis_index("s") * NC + lax.axis_index("c")
        base = wid * n
        lanes = lax.iota(jnp.int32, _L)

        def splat(x):
            return jnp.full((_L,), x, jnp.int32)

        def sread(ref1d, e):
            v = plsc.load_gather(ref1d, [splat(e)])
            return lax.reduce_max(v, (0,))

        def side(keys_hbm, tab, gath):
            pltpu.sync_copy(keys_hbm.at[pl.ds(base, n)], keys_v)

            for v in range(n // _L):
                k = keys_v[pl.ds(v * _L, _L)]
                tgs_v[pl.ds(v * _L, _L)] = lax.shift_right_logical(k, _BSH)
            for v in range(_RUNS_MAX // _L):
                runs_v[pl.ds(v * _L, _L)] = splat(n)

            def pa(v, cnt_vec):
                tg = tgs_v[pl.ds(v * _L, _L)]
                pidx = jnp.maximum(v * _L - 1 + lanes, 0)
                prev = plsc.load_gather(tgs_v, [pidx])
                m = tg != prev
                m = jnp.logical_or(m, jnp.logical_and(v == 0, lanes == 0))
                mi = m.astype(jnp.int32)
                excl = plsc.cumsum(mi) - mi
                plsc.store_scatter(runs_v, [cnt_vec + excl], v * _L + lanes,
                                   mask=m)
                return cnt_vec + plsc.all_reduce_population_count(m)
            cnt_vec = lax.fori_loop(0, n // _L, pa, jnp.zeros((_L,), jnp.int32))
            cnt = lax.reduce_max(cnt_vec, (0,))
            cnt2 = (cnt + 1) // 2

            def fetch(e0c, blk, sem):
                tg = sread(tgs_v, e0c)
                off = pl.multiple_of(
                    jnp.minimum(tg * _BW, vpad - _BW), _W)
                return pltpu.async_copy(tab.at[:, pl.ds(off, _BW)], blk, sem)

            def extract(e0, e1, blk):

                def elem(e, c):
                    kv = plsc.load_gather(keys_v, [splat(e)])
                    boff = jnp.minimum(
                        lax.shift_right_logical(kv, _BSH) * _BW, vpad - _BW)
                    lane = kv - boff
                    for h in range(NH):
                        v = plsc.load_gather(blk, [lanes + _L * h, lane])
                        staged[e, pl.ds(_L * h, _L)] = v
                    return c
                lax.fori_loop(e0, e1, elem, 0)

            NG = 6

            def kb(k, c):
                j = NG * k
                es = [sread(runs_v, j + t) for t in range(NG + 1)]
                e0c = [jnp.minimum(e, n - 1) for e in es[:NG]]
                hs = [fetch(e0c[t], bufs[t], semA) for t in range(NG)]
                for t in range(NG):
                    hs[t].wait()
                for t in range(NG):
                    extract(es[t], jnp.minimum(es[t + 1], n), bufs[t])
                return c
            lax.fori_loop(0, (cnt + NG - 1) // NG, kb, 0)
            pltpu.sync_copy(staged, gath.at[pl.ds(base, n)])

        side(uk_hbm, u_tab_hbm, u_gath)
        side(ik_hbm, i_tab_hbm, i_gath)

    return stage_kernel(uk, ik, u_tab, i_tab)


def _dot(u_gath, i_gath, inv_u, inv_i, D):
    B = u_gath.shape[0]
    info = plsc.get_sparse_core_info()
    NC, NS = info.num_cores, info.num_subcores
    NW = NC * NS
    n = B // NW
    half = n // 2
    mesh = plsc.VectorSubcoreMesh(core_axis_name="c", subcore_axis_name="s")

    @functools.partial(
        pl.kernel,
        out_type=jax.ShapeDtypeStruct((B,), jnp.float32),
        mesh=mesh,
        scratch_types=[
            pltpu.VMEM((half,), jnp.int32),
            pltpu.VMEM((half,), jnp.int32),
            pltpu.VMEM((half, _W), jnp.float32),
            pltpu.VMEM((half, _W), jnp.float32),
            pltpu.VMEM((n,), jnp.float32),
            pltpu.SemaphoreType.DMA,
            pltpu.SemaphoreType.DMA,
        ],
        compiler_params=pltpu.CompilerParams(
            needs_layout_passes=False, use_tc_tiling_on_sc=True,
            disable_bounds_checks=True),
    )
    def dot_kernel(u_gath_hbm, i_gath_hbm, inv_u_hbm, inv_i_hbm, out_hbm,
                   iu_v, ii_v, u_buf, i_buf, out_v, semu, semi):
        wid = lax.axis_index("s") * NC + lax.axis_index("c")
        base = wid * n
        lanes = lax.iota(jnp.int32, _L)

        for hb in range(2):
            off = base + hb * half
            pltpu.sync_copy(inv_u_hbm.at[pl.ds(off, half)], iu_v)
            pltpu.sync_copy(inv_i_hbm.at[pl.ds(off, half)], ii_v)
            cu = pltpu.async_copy(u_gath_hbm.at[iu_v], u_buf, semu)
            ci = pltpu.async_copy(i_gath_hbm.at[ii_v], i_buf, semi)
            cu.wait()
            ci.wait()

            def group(g, c):
                rows = g * _L + lanes
                acc = jnp.zeros((_L,), jnp.float32)
                for d in range(D):
                    col = jnp.full((_L,), d, jnp.int32)
                    uu = plsc.load_gather(u_buf, [rows, col])
                    ii = plsc.load_gather(i_buf, [rows, col])
                    acc = acc + uu * ii
                out_v[pl.ds(hb * half + g * _L, _L)] = acc
                return c
            lax.fori_loop(0, half // _L, group, 0)
        pltpu.sync_copy(out_v, out_hbm.at[pl.ds(base, n)])

    return dot_kernel(u_gath, i_gath, inv_u, inv_i)


def kernel(u_id, i_id, user_emb, item_emb, user_bias, item_bias):
    del user_bias, item_bias
    u_id = u_id.astype(jnp.int32)
    i_id = i_id.astype(jnp.int32)
    B = u_id.shape[0]
    D = user_emb.shape[1]
    pos = lax.iota(jnp.int32, B)
    uk, up = lax.sort((u_id, pos), num_keys=1)
    ik, ip = lax.sort((i_id, pos), num_keys=1)
    inv_u = lax.sort((up, pos), num_keys=1)[1]
    inv_i = lax.sort((ip, pos), num_keys=1)[1]
    u_gath, i_gath = _stage_pair(uk, ik, user_emb.T, item_emb.T)
    return _dot(u_gath, i_gath, inv_u, inv_i, D)


# speedup vs baseline: 1.0706x; 1.0009x over previous
import functools

import jax
import jax.numpy as jnp
from jax import lax
from jax.experimental import pallas as pl
from jax.experimental.pallas import tpu as pltpu
from jax.experimental.pallas import tpu_sc as plsc

_L = 16
_W = 128
_BW = 128
_BSH = 7
_RUNS_MAX = 544


def _stage_pair(uk, ik, u_tab, i_tab):
    B = uk.shape[0]
    D, V = u_tab.shape
    info = plsc.get_sparse_core_info()
    NC, NS = info.num_cores, info.num_subcores
    NW = NC * NS
    n = B // NW
    NH = D // _L
    mesh = plsc.VectorSubcoreMesh(core_axis_name="c", subcore_axis_name="s")

    @functools.partial(
        pl.kernel,
        out_type=(jax.ShapeDtypeStruct((B, _W), jnp.float32),
                  jax.ShapeDtypeStruct((B, _W), jnp.float32)),
        mesh=mesh,
        scratch_types=[
            pltpu.VMEM((n,), jnp.int32),
            pltpu.VMEM((n,), jnp.int32),
            pltpu.VMEM((_RUNS_MAX,), jnp.int32),
            pltpu.VMEM((D, _BW), jnp.float32),
            pltpu.VMEM((D, _BW), jnp.float32),
            pltpu.VMEM((D, _BW), jnp.float32),
            pltpu.VMEM((D, _BW), jnp.float32),
            pltpu.VMEM((D, _BW), jnp.float32),
            pltpu.VMEM((D, _BW), jnp.float32),
            pltpu.VMEM((n, _W), jnp.float32),
            pltpu.SemaphoreType.DMA,
        ],
        compiler_params=pltpu.CompilerParams(
            needs_layout_passes=False, use_tc_tiling_on_sc=True,
            disable_bounds_checks=True),
    )
    def stage_kernel(uk_hbm, ik_hbm, u_tab_hbm, i_tab_hbm, u_gath, i_gath,
                     keys_v, tgs_v, runs_v, bb0, bb1, bb2, bb3, bb4, bb5,
                     staged, semA):
        bufs = (bb0, bb1, bb2, bb3, bb4, bb5)
        vpad = -(-V // _W) * _W
        wid = lax.axis_index("s") * NC + lax.axis_index("c")
        base = wid * n
        lanes = lax.iota(jnp.int32, _L)

        def splat(x):
            return jnp.full((_L,), x, jnp.int32)

        def sread(ref1d, e):
            v = plsc.load_gather(ref1d, [splat(e)])
            return lax.reduce_max(v, (0,))

        def side(keys_hbm, tab, gath):
            pltpu.sync_copy(keys_hbm.at[pl.ds(base, n)], keys_v)

            for v in range(n // _L):
                k = keys_v[pl.ds(v * _L, _L)]
                tgs_v[pl.ds(v * _L, _L)] = lax.shift_right_logical(k, _BSH)
            for v in range(_RUNS_MAX // _L):
                runs_v[pl.ds(v * _L, _L)] = splat(n)

            def pa(v, cnt_vec):
                tg = tgs_v[pl.ds(v * _L, _L)]
                pidx = jnp.maximum(v * _L - 1 + lanes, 0)
                prev = plsc.load_gather(tgs_v, [pidx])
                m = tg != prev
                m = jnp.logical_or(m, jnp.logical_and(v == 0, lanes == 0))
                mi = m.astype(jnp.int32)
                excl = plsc.cumsum(mi) - mi
                plsc.store_scatter(runs_v, [cnt_vec + excl], v * _L + lanes,
                                   mask=m)
                return cnt_vec + plsc.all_reduce_population_count(m)
            cnt_vec = lax.fori_loop(0, n // _L, pa, jnp.zeros((_L,), jnp.int32))
            cnt = lax.reduce_max(cnt_vec, (0,))
            cnt2 = (cnt + 1) // 2

            def fetch(e0c, blk, sem):
                tg = sread(tgs_v, e0c)
                off = pl.multiple_of(
                    jnp.minimum(tg * _BW, vpad - _BW), _W)
                return pltpu.async_copy(tab.at[:, pl.ds(off, _BW)], blk, sem)

            def extract(e0, e1, blk):

                def elem(e, c):
                    kv = plsc.load_gather(keys_v, [splat(e)])
                    boff = jnp.minimum(
                        lax.shift_right_logical(kv, _BSH) * _BW, vpad - _BW)
                    lane = kv - boff
                    for h in range(NH):
                        v = plsc.load_gather(blk, [lanes + _L * h, lane])
                        staged[e, pl.ds(_L * h, _L)] = v
                    return c
                lax.fori_loop(e0, e1, elem, 0)

            NG = 6

            def kb(k, c):
                j = NG * k
                es = [sread(runs_v, j + t) for t in range(NG + 1)]
                e0c = [jnp.minimum(e, n - 1) for e in es[:NG]]
                hs = [fetch(e0c[t], bufs[t], semA) for t in range(NG)]
                for t in range(NG):
                    hs[t].wait()
                for t in range(NG):
                    extract(es[t], jnp.minimum(es[t + 1], n), bufs[t])
                return c
            lax.fori_loop(0, (cnt + NG - 1) // NG, kb, 0)
            pltpu.sync_copy(staged, gath.at[pl.ds(base, n)])

        side(uk_hbm, u_tab_hbm, u_gath)
        side(ik_hbm, i_tab_hbm, i_gath)

    return stage_kernel(uk, ik, u_tab, i_tab)


def _dot(u_gath, i_gath, inv_u, inv_i, D):
    B = u_gath.shape[0]
    info = plsc.get_sparse_core_info()
    NC, NS = info.num_cores, info.num_subcores
    NW = NC * NS
    n = B // NW
    half = n // 2
    mesh = plsc.VectorSubcoreMesh(core_axis_name="c", subcore_axis_name="s")

    @functools.partial(
        pl.kernel,
        out_type=jax.ShapeDtypeStruct((B,), jnp.float32),
        mesh=mesh,
        scratch_types=[
            pltpu.VMEM((half,), jnp.int32),
            pltpu.VMEM((half,), jnp.int32),
            pltpu.VMEM((half, _W), jnp.float32),
            pltpu.VMEM((half, _W), jnp.float32),
            pltpu.VMEM((n,), jnp.float32),
            pltpu.SemaphoreType.DMA,
            pltpu.SemaphoreType.DMA,
        ],
        compiler_params=pltpu.CompilerParams(
            needs_layout_passes=False, use_tc_tiling_on_sc=True,
            disable_bounds_checks=True),
    )
    def dot_kernel(u_gath_hbm, i_gath_hbm, inv_u_hbm, inv_i_hbm, out_hbm,
                   iu_v, ii_v, u_buf, i_buf, out_v, semu, semi):
        wid = lax.axis_index("s") * NC + lax.axis_index("c")
        base = wid * n
        lanes = lax.iota(jnp.int32, _L)

        for hb in range(2):
            off = base + hb * half
            pltpu.sync_copy(inv_u_hbm.at[pl.ds(off, half)], iu_v)
            pltpu.sync_copy(inv_i_hbm.at[pl.ds(off, half)], ii_v)
            cu = pltpu.async_copy(u_gath_hbm.at[iu_v], u_buf, semu)
            ci = pltpu.async_copy(i_gath_hbm.at[ii_v], i_buf, semi)
            cu.wait()
            ci.wait()

            def group(g, c):
                rows = g * _L + lanes
                acc = jnp.zeros((_L,), jnp.float32)
                for d in range(D):
                    col = jnp.full((_L,), d, jnp.int32)
                    uu = plsc.load_gather(u_buf, [rows, col])
                    ii = plsc.load_gather(i_buf, [rows, col])
                    acc = acc + uu * ii
                out_v[pl.ds(hb * half + g * _L, _L)] = acc
                return c
            lax.fori_loop(0, half // _L, group, 0)
        pltpu.sync_copy(out_v, out_hbm.at[pl.ds(base, n)])

    return dot_kernel(u_gath, i_gath, inv_u, inv_i)


def kernel(u_id, i_id, user_emb, item_emb, user_bias, item_bias):
    del user_bias, item_bias
    u_id = u_id.astype(jnp.int32)
    i_id = i_id.astype(jnp.int32)
    B = u_id.shape[0]
    D = user_emb.shape[1]
    pos = lax.iota(jnp.int32, B)
    uk, up = lax.sort((u_id, pos), num_keys=1)
    ik, ip = lax.sort((i_id, pos), num_keys=1)
    inv_u = jnp.zeros((B,), jnp.int32).at[up].set(pos, unique_indices=True)
    inv_i = jnp.zeros((B,), jnp.int32).at[ip].set(pos, unique_indices=True)
    u_gath, i_gath = _stage_pair(uk, ik, user_emb.T, item_emb.T)
    return _dot(u_gath, i_gath, inv_u, inv_i, D)


# speedup vs baseline: 1.0964x; 1.0241x over previous
import functools

import jax
import jax.numpy as jnp
from jax import lax
from jax.experimental import pallas as pl
from jax.experimental.pallas import tpu as pltpu
from jax.experimental.pallas import tpu_sc as plsc

_L = 16
_W = 128
_BW = 128
_BSH = 7
_RUNS_MAX = 544


def _stage_pair(uk, ik, u_tab, i_tab):
    B = uk.shape[0]
    D, V = u_tab.shape
    info = plsc.get_sparse_core_info()
    NC, NS = info.num_cores, info.num_subcores
    NW = NC * NS
    n = B // NW
    NH = D // _L
    mesh = plsc.VectorSubcoreMesh(core_axis_name="c", subcore_axis_name="s")

    @functools.partial(
        pl.kernel,
        out_type=(jax.ShapeDtypeStruct((B, _W), jnp.float32),
                  jax.ShapeDtypeStruct((B, _W), jnp.float32)),
        mesh=mesh,
        scratch_types=[
            pltpu.VMEM((n,), jnp.int32),
            pltpu.VMEM((n,), jnp.int32),
            pltpu.VMEM((_RUNS_MAX,), jnp.int32),
            pltpu.VMEM((D, _BW), jnp.float32),
            pltpu.VMEM((D, _BW), jnp.float32),
            pltpu.VMEM((D, _BW), jnp.float32),
            pltpu.VMEM((D, _BW), jnp.float32),
            pltpu.VMEM((D, _BW), jnp.float32),
            pltpu.VMEM((D, _BW), jnp.float32),
            pltpu.VMEM((D, _BW), jnp.float32),
            pltpu.VMEM((n, _W), jnp.float32),
            pltpu.SemaphoreType.DMA,
        ],
        compiler_params=pltpu.CompilerParams(
            needs_layout_passes=False, use_tc_tiling_on_sc=True,
            disable_bounds_checks=True),
    )
    def stage_kernel(uk_hbm, ik_hbm, u_tab_hbm, i_tab_hbm, u_gath, i_gath,
                     keys_v, tgs_v, runs_v, bb0, bb1, bb2, bb3, bb4, bb5, bb6,
                     staged, semA):
        bufs = (bb0, bb1, bb2, bb3, bb4, bb5, bb6)
        vpad = -(-V // _W) * _W
        wid = lax.axis_index("s") * NC + lax.axis_index("c")
        base = wid * n
        lanes = lax.iota(jnp.int32, _L)

        def splat(x):
            return jnp.full((_L,), x, jnp.int32)

        def sread(ref1d, e):
            v = plsc.load_gather(ref1d, [splat(e)])
            return lax.reduce_max(v, (0,))

        def side(keys_hbm, tab, gath):
            pltpu.sync_copy(keys_hbm.at[pl.ds(base, n)], keys_v)

            for v in range(n // _L):
                k = keys_v[pl.ds(v * _L, _L)]
                tgs_v[pl.ds(v * _L, _L)] = lax.shift_right_logical(k, _BSH)
            for v in range(_RUNS_MAX // _L):
                runs_v[pl.ds(v * _L, _L)] = splat(n)

            def pa(v, cnt_vec):
                tg = tgs_v[pl.ds(v * _L, _L)]
                pidx = jnp.maximum(v * _L - 1 + lanes, 0)
                prev = plsc.load_gather(tgs_v, [pidx])
                m = tg != prev
                m = jnp.logical_or(m, jnp.logical_and(v == 0, lanes == 0))
                mi = m.astype(jnp.int32)
                excl = plsc.cumsum(mi) - mi
                plsc.store_scatter(runs_v, [cnt_vec + excl], v * _L + lanes,
                                   mask=m)
                return cnt_vec + plsc.all_reduce_population_count(m)
            cnt_vec = lax.fori_loop(0, n // _L, pa, jnp.zeros((_L,), jnp.int32))
            cnt = lax.reduce_max(cnt_vec, (0,))
            cnt2 = (cnt + 1) // 2

            def fetch(e0c, blk, sem):
                tg = sread(tgs_v, e0c)
                off = pl.multiple_of(
                    jnp.minimum(tg * _BW, vpad - _BW), _W)
                return pltpu.async_copy(tab.at[:, pl.ds(off, _BW)], blk, sem)

            def extract(e0, e1, blk):

                def elem(e, c):
                    kv = plsc.load_gather(keys_v, [splat(e)])
                    boff = jnp.minimum(
                        lax.shift_right_logical(kv, _BSH) * _BW, vpad - _BW)
                    lane = kv - boff
                    for h in range(NH):
                        v = plsc.load_gather(blk, [lanes + _L * h, lane])
                        staged[e, pl.ds(_L * h, _L)] = v
                    return c
                lax.fori_loop(e0, e1, elem, 0)

            NG = 7

            def kb(k, c):
                j = NG * k
                es = [sread(runs_v, j + t) for t in range(NG + 1)]
                e0c = [jnp.minimum(e, n - 1) for e in es[:NG]]
                hs = [fetch(e0c[t], bufs[t], semA) for t in range(NG)]
                for t in range(NG):
                    hs[t].wait()
                for t in range(NG):
                    extract(es[t], jnp.minimum(es[t + 1], n), bufs[t])
                return c
            lax.fori_loop(0, (cnt + NG - 1) // NG, kb, 0)
            pltpu.sync_copy(staged, gath.at[pl.ds(base, n)])

        side(uk_hbm, u_tab_hbm, u_gath)
        side(ik_hbm, i_tab_hbm, i_gath)

    return stage_kernel(uk, ik, u_tab, i_tab)


def _dot(u_gath, i_gath, inv_u, inv_i, D):
    B = u_gath.shape[0]
    info = plsc.get_sparse_core_info()
    NC, NS = info.num_cores, info.num_subcores
    NW = NC * NS
    n = B // NW
    half = n // 2
    mesh = plsc.VectorSubcoreMesh(core_axis_name="c", subcore_axis_name="s")

    @functools.partial(
        pl.kernel,
        out_type=jax.ShapeDtypeStruct((B,), jnp.float32),
        mesh=mesh,
        scratch_types=[
            pltpu.VMEM((half,), jnp.int32),
            pltpu.VMEM((half,), jnp.int32),
            pltpu.VMEM((half, _W), jnp.float32),
            pltpu.VMEM((half, _W), jnp.float32),
            pltpu.VMEM((n,), jnp.float32),
            pltpu.SemaphoreType.DMA,
            pltpu.SemaphoreType.DMA,
        ],
        compiler_params=pltpu.CompilerParams(
            needs_layout_passes=False, use_tc_tiling_on_sc=True,
            disable_bounds_checks=True),
    )
    def dot_kernel(u_gath_hbm, i_gath_hbm, inv_u_hbm, inv_i_hbm, out_hbm,
                   iu_v, ii_v, u_buf, i_buf, out_v, semu, semi):
        wid = lax.axis_index("s") * NC + lax.axis_index("c")
        base = wid * n
        lanes = lax.iota(jnp.int32, _L)

        for hb in range(2):
            off = base + hb * half
            pltpu.sync_copy(inv_u_hbm.at[pl.ds(off, half)], iu_v)
            pltpu.sync_copy(inv_i_hbm.at[pl.ds(off, half)], ii_v)
            cu = pltpu.async_copy(u_gath_hbm.at[iu_v], u_buf, semu)
            ci = pltpu.async_copy(i_gath_hbm.at[ii_v], i_buf, semi)
            cu.wait()
            ci.wait()

            def group(g, c):
                rows = g * _L + lanes
                acc = jnp.zeros((_L,), jnp.float32)
                for d in range(D):
                    col = jnp.full((_L,), d, jnp.int32)
                    uu = plsc.load_gather(u_buf, [rows, col])
                    ii = plsc.load_gather(i_buf, [rows, col])
                    acc = acc + uu * ii
                out_v[pl.ds(hb * half + g * _L, _L)] = acc
                return c
            lax.fori_loop(0, half // _L, group, 0)
        pltpu.sync_copy(out_v, out_hbm.at[pl.ds(base, n)])

    return dot_kernel(u_gath, i_gath, inv_u, inv_i)


def kernel(u_id, i_id, user_emb, item_emb, user_bias, item_bias):
    del user_bias, item_bias
    u_id = u_id.astype(jnp.int32)
    i_id = i_id.astype(jnp.int32)
    B = u_id.shape[0]
    D = user_emb.shape[1]
    pos = lax.iota(jnp.int32, B)
    uk, up = lax.sort((u_id, pos), num_keys=1)
    ik, ip = lax.sort((i_id, pos), num_keys=1)
    inv_u = jnp.zeros((B,), jnp.int32).at[up].set(pos, unique_indices=True)
    inv_i = jnp.zeros((B,), jnp.int32).at[ip].set(pos, unique_indices=True)
    u_gath, i_gath = _stage_pair(uk, ik, user_emb.T, item_emb.T)
    return _dot(u_gath, i_gath, inv_u, inv_i, D)
